# Initial kernel scaffold; baseline (speedup 1.0000x reference)
import jax
import jax.numpy as jnp
from jax.experimental import pallas as pl


def kernel(x, adj, W1, b1, W2, b2, W3, b3):
    raise NotImplementedError("write your pallas kernel here")


# speedup vs baseline: 6.4088x; 6.4088x over previous
import functools

import jax
import jax.numpy as jnp
from jax import lax
from jax.experimental import pallas as pl
from jax.experimental.pallas import tpu as pltpu
from jax.experimental.pallas import tpu_sc as plsc

N = 10000
E = 320000
NFEAT = 128
NHID = 128
NCLASS = 40
NCLS_PAD = 48

NPAD = 10240
CHUNK = 128
N_TILES = 32
CHUNKS_PER_TILE = 79
N_CHUNKS = N_TILES * CHUNKS_PER_TILE
E_PAD = N_CHUNKS * CHUNK
ROWS_PER_TILE = NPAD // 16


def _make_spmm(D):
  mesh = plsc.VectorSubcoreMesh(core_axis_name="c", subcore_axis_name="s")
  extra = {}
  if D % 128 != 0:
    extra = dict(compiler_params=pltpu.CompilerParams(
        use_tc_tiling_on_sc=False))

  @functools.partial(
      pl.kernel,
      mesh=mesh,
      out_type=jax.ShapeDtypeStruct((2 * NPAD, D), jnp.float32),
      **extra,
      scratch_types=[
          pltpu.VMEM((CHUNK,), jnp.int32),
          pltpu.VMEM((CHUNK,), jnp.int32),
          pltpu.VMEM((CHUNK, D), jnp.float32),
          pltpu.VMEM((CHUNK, D), jnp.float32),
          pltpu.VMEM_SHARED((NPAD, D), jnp.float32),
          pltpu.SemaphoreType.DMA,
      ],
  )
  def spmm(support_hbm, src_hbm, dst_hbm, out_hbm, sidx, didx, rows, zbuf,
           acc, sem):
    c = lax.axis_index("c")
    s = lax.axis_index("s")
    wid = c * 16 + s

    zero16 = jnp.zeros((16,), jnp.float32)

    def zrow(i, carry):
      for l in range(D // 16):
        zbuf[i, pl.ds(l * 16, 16)] = zero16
      return carry

    lax.fori_loop(0, CHUNK, zrow, 0)
    row0 = s * ROWS_PER_TILE
    for r in range(ROWS_PER_TILE // CHUNK):
      pltpu.sync_copy(zbuf, acc.at[pl.ds(row0 + r * CHUNK, CHUNK)])
    plsc.subcore_barrier()

    base_chunk = wid * CHUNKS_PER_TILE

    def body(j, carry):
      chunk = base_chunk + j
      pltpu.sync_copy(src_hbm.at[chunk], sidx)
      pltpu.sync_copy(dst_hbm.at[chunk], didx)
      pltpu.async_copy(support_hbm.at[sidx], rows, sem).wait()
      pltpu.sync_copy(rows, acc.at[didx], add=True)
      return carry

    lax.fori_loop(0, CHUNKS_PER_TILE, body, 0)
    plsc.subcore_barrier()

    pltpu.sync_copy(
        acc.at[pl.ds(row0, ROWS_PER_TILE)],
        out_hbm.at[pl.ds(c * NPAD + row0, ROWS_PER_TILE)])

  return spmm


_spmm_128 = _make_spmm(NHID)
_spmm_48 = _make_spmm(NCLS_PAD)


def _mm(x, W, rows_blk=1000):
  n, k = x.shape
  m = W.shape[1]

  def kern(x_ref, w_ref, o_ref):
    o_ref[...] = jnp.dot(x_ref[...], w_ref[...],
                         preferred_element_type=jnp.float32)

  return pl.pallas_call(
      kern,
      grid=(n // rows_blk,),
      in_specs=[
          pl.BlockSpec((rows_blk, k), lambda i: (i, 0)),
          pl.BlockSpec((k, m), lambda i: (0, 0)),
      ],
      out_specs=pl.BlockSpec((rows_blk, m), lambda i: (i, 0)),
      out_shape=jax.ShapeDtypeStruct((n, m), jnp.float32),
  )(x, W)


def _combine_mm(p0, p1, b, W, rows_blk=1000):
  n, k = p0.shape
  m = W.shape[1]

  def kern(p0_ref, p1_ref, b_ref, w_ref, o_ref):
    h = jnp.maximum(p0_ref[...] + p1_ref[...] + b_ref[...], 0.0)
    o_ref[...] = jnp.dot(h, w_ref[...], preferred_element_type=jnp.float32)

  return pl.pallas_call(
      kern,
      grid=(n // rows_blk,),
      in_specs=[
          pl.BlockSpec((rows_blk, k), lambda i: (i, 0)),
          pl.BlockSpec((rows_blk, k), lambda i: (i, 0)),
          pl.BlockSpec((1, k), lambda i: (0, 0)),
          pl.BlockSpec((k, m), lambda i: (0, 0)),
      ],
      out_specs=pl.BlockSpec((rows_blk, m), lambda i: (i, 0)),
      out_shape=jax.ShapeDtypeStruct((n, m), jnp.float32),
  )(p0, p1, b.reshape(1, k), W)


def _final_logsoftmax(p0, p1, b, rows_blk=1000):
  n, k = p0.shape

  def kern(p0_ref, p1_ref, b_ref, o_ref):
    v = p0_ref[...] + p1_ref[...] + b_ref[...]
    col = lax.broadcasted_iota(jnp.int32, v.shape, 1)
    valid = col < NCLASS
    vm = jnp.where(valid, v, jnp.float32(-1e30))
    mx = jnp.max(vm, axis=1, keepdims=True)
    ex = jnp.where(valid, jnp.exp(v - mx), 0.0)
    ssum = jnp.sum(ex, axis=1, keepdims=True)
    o_ref[...] = v - mx - jnp.log(ssum)

  out = pl.pallas_call(
      kern,
      grid=(n // rows_blk,),
      in_specs=[
          pl.BlockSpec((rows_blk, k), lambda i: (i, 0)),
          pl.BlockSpec((rows_blk, k), lambda i: (i, 0)),
          pl.BlockSpec((1, k), lambda i: (0, 0)),
      ],
      out_specs=pl.BlockSpec((rows_blk, k), lambda i: (i, 0)),
      out_shape=jax.ShapeDtypeStruct((n, k), jnp.float32),
  )(p0, p1, b.reshape(1, k))
  return out[:, :NCLASS]


def kernel(x, adj, W1, b1, W2, b2, W3, b3):
  pad = jnp.arange(E_PAD - E, dtype=jnp.int32)
  src = jnp.concatenate([adj[0], pad % N]).reshape(N_CHUNKS, CHUNK)
  dst = jnp.concatenate([adj[1], N + pad % (NPAD - N)]).reshape(
      N_CHUNKS, CHUNK)

  support = _mm(x, W1)
  parts = _spmm_128(support, src, dst)
  p0, p1 = parts[:N], parts[NPAD:NPAD + N]

  support = _combine_mm(p0, p1, b1, W2)
  parts = _spmm_128(support, src, dst)
  p0, p1 = parts[:N], parts[NPAD:NPAD + N]

  W3p = jnp.concatenate(
      [W3, jnp.zeros((NHID, NCLS_PAD - NCLASS), jnp.float32)], axis=1)
  b3p = jnp.concatenate([b3, jnp.zeros((NCLS_PAD - NCLASS,), jnp.float32)])
  support = _combine_mm(p0, p1, b2, W3p)
  parts = _spmm_48(support, src, dst)
  p0, p1 = parts[:N], parts[NPAD:NPAD + N]

  return _final_logsoftmax(p0, p1, b3p)


# speedup vs baseline: 8.2196x; 1.2825x over previous
import functools

import jax
import jax.numpy as jnp
from jax import lax
from jax.experimental import pallas as pl
from jax.experimental.pallas import tpu as pltpu
from jax.experimental.pallas import tpu_sc as plsc

N = 10000
E = 320000
NFEAT = 128
NHID = 128
NCLASS = 40
NCLS_PAD = 48
HALF = NHID // 2

CHUNK_A = 128
CPT_A = 160
E_PAD_A = 16 * CPT_A * CHUNK_A
NPAD_A = 10240
RPT_A = NPAD_A // 16

CHUNK_B = 96
CPT_B = 106
E_PAD_B = 32 * CPT_B * CHUNK_B
NPAD_B = 10112
RPT_B = NPAD_B // 16


def _zero_acc_slice(rows0, acc, row0, rpt, chunk, d):
  zero16 = jnp.zeros((16,), jnp.float32)

  def zrow(i, carry):
    for l in range(d // 16):
      rows0[i, pl.ds(l * 16, 16)] = zero16
    return carry

  lax.fori_loop(0, chunk, zrow, 0)
  nfull = rpt // chunk
  for r in range(nfull):
    pltpu.sync_copy(rows0, acc.at[pl.ds(row0 + r * chunk, chunk)])
  rem = rpt - nfull * chunk
  if rem:
    pltpu.sync_copy(rows0.at[pl.ds(0, rem)],
                    acc.at[pl.ds(row0 + nfull * chunk, rem)])


def _pipeline(support_hbm, acc, sidx, didx, rows, gsem, ssem, cpt):
  def gather(jj, b):
    pltpu.async_copy(support_hbm.at[sidx.at[jj]], rows[b], gsem[b])

  def wait_gather(jj, b):
    pltpu.make_async_copy(support_hbm.at[sidx.at[jj]], rows[b],
                          gsem[b]).wait()

  def scatter(jj, b):
    pltpu.async_copy(rows[b], acc.at[didx.at[jj]], ssem[b], add=True)

  def wait_scatter(jj, b):
    pltpu.make_async_copy(rows[b], acc.at[didx.at[jj]], ssem[b]).wait()

  gather(0, 0)

  def body(j2, carry):
    for b in range(2):
      jj = j2 * 2 + b
      wait_gather(jj, b)
      scatter(jj, b)

      @pl.when(jj >= 1)
      def _():
        wait_scatter(jj - 1, 1 - b)

      @pl.when(jj + 1 < cpt)
      def _():
        gather(jj + 1, 1 - b)
    return carry

  lax.fori_loop(0, cpt // 2, body, 0)
  wait_scatter(cpt - 1, (cpt - 1) % 2)


def _make_spmm_col():
  mesh = plsc.VectorSubcoreMesh(core_axis_name="c", subcore_axis_name="s")

  @functools.partial(
      pl.kernel,
      mesh=mesh,
      out_type=jax.ShapeDtypeStruct((2 * NPAD_A, HALF), jnp.float32),
      compiler_params=pltpu.CompilerParams(use_tc_tiling_on_sc=False),
      scratch_types=[
          pltpu.VMEM((CPT_A, CHUNK_A), jnp.int32),
          pltpu.VMEM((CPT_A, CHUNK_A), jnp.int32),
          [pltpu.VMEM((CHUNK_A, HALF), jnp.float32) for _ in range(2)],
          pltpu.VMEM_SHARED((NPAD_A, HALF), jnp.float32),
          [pltpu.SemaphoreType.DMA for _ in range(2)],
          [pltpu.SemaphoreType.DMA for _ in range(2)],
          pltpu.SemaphoreType.DMA,
      ],
  )
  def spmm(table_hbm, src_hbm, dst_hbm, out_hbm, sidx, didx, rows, acc,
           gsem, ssem, isem):
    c = lax.axis_index("c")
    s = lax.axis_index("s")
    icp0 = pltpu.async_copy(src_hbm.at[c, s], sidx, isem)
    icp1 = pltpu.async_copy(dst_hbm.at[s], didx, isem)
    row0 = s * RPT_A
    _zero_acc_slice(rows[0], acc, row0, RPT_A, CHUNK_A, HALF)
    icp0.wait()
    icp1.wait()
    plsc.subcore_barrier()
    _pipeline(table_hbm, acc, sidx, didx, rows, gsem, ssem, CPT_A)
    plsc.subcore_barrier()
    pltpu.sync_copy(acc.at[pl.ds(row0, RPT_A)],
                    out_hbm.at[pl.ds(c * NPAD_A + row0, RPT_A)])

  return spmm


def _make_spmm_edge():
  mesh = plsc.VectorSubcoreMesh(core_axis_name="c", subcore_axis_name="s")

  @functools.partial(
      pl.kernel,
      mesh=mesh,
      out_type=jax.ShapeDtypeStruct((2 * NPAD_B, NCLS_PAD), jnp.float32),
      compiler_params=pltpu.CompilerParams(use_tc_tiling_on_sc=False),
      scratch_types=[
          pltpu.VMEM((CPT_B, CHUNK_B), jnp.int32),
          pltpu.VMEM((CPT_B, CHUNK_B), jnp.int32),
          [pltpu.VMEM((CHUNK_B, NCLS_PAD), jnp.float32) for _ in range(2)],
          pltpu.VMEM_SHARED((NPAD_B, NCLS_PAD), jnp.float32),
          [pltpu.SemaphoreType.DMA for _ in range(2)],
          [pltpu.SemaphoreType.DMA for _ in range(2)],
          pltpu.SemaphoreType.DMA,
      ],
  )
  def spmm(support_hbm, src_hbm, dst_hbm, out_hbm, sidx, didx, rows, acc,
           gsem, ssem, isem):
    c = lax.axis_index("c")
    s = lax.axis_index("s")
    wid = c * 16 + s
    icp0 = pltpu.async_copy(src_hbm.at[wid], sidx, isem)
    icp1 = pltpu.async_copy(dst_hbm.at[wid], didx, isem)
    row0 = s * RPT_B
    _zero_acc_slice(rows[0], acc, row0, RPT_B, CHUNK_B, NCLS_PAD)
    icp0.wait()
    icp1.wait()
    plsc.subcore_barrier()
    _pipeline(support_hbm, acc, sidx, didx, rows, gsem, ssem, CPT_B)
    plsc.subcore_barrier()
    pltpu.sync_copy(acc.at[pl.ds(row0, RPT_B)],
                    out_hbm.at[pl.ds(c * NPAD_B + row0, RPT_B)])

  return spmm


_spmm_col = _make_spmm_col()
_spmm_edge = _make_spmm_edge()


def _mm_split(x, W, rows_blk=1000):
  n, k = x.shape

  def kern(x_ref, w_ref, o_ref):
    o_ref[...] = jnp.dot(x_ref[...], w_ref[0],
                         preferred_element_type=jnp.float32)[None]

  return pl.pallas_call(
      kern,
      grid=(n // rows_blk, 2),
      in_specs=[
          pl.BlockSpec((rows_blk, k), lambda i, j: (i, 0)),
          pl.BlockSpec((1, k, HALF), lambda i, j: (j, 0, 0)),
      ],
      out_specs=pl.BlockSpec((1, rows_blk, HALF), lambda i, j: (j, i, 0)),
      out_shape=jax.ShapeDtypeStruct((2, n, HALF), jnp.float32),
  )(x, W)


def _combine_mm_split(h0, h1, b, W, rows_blk=1000):
  n = h0.shape[0]
  k = W.shape[1]

  def kern(h0_ref, h1_ref, b_ref, w_ref, o_ref):
    h = jnp.concatenate([h0_ref[...], h1_ref[...]], axis=1)
    h = jnp.maximum(h + b_ref[...], 0.0)
    o_ref[...] = jnp.dot(h, w_ref[0],
                         preferred_element_type=jnp.float32)[None]

  return pl.pallas_call(
      kern,
      grid=(n // rows_blk, 2),
      in_specs=[
          pl.BlockSpec((rows_blk, HALF), lambda i, j: (i, 0)),
          pl.BlockSpec((rows_blk, HALF), lambda i, j: (i, 0)),
          pl.BlockSpec((1, k), lambda i, j: (0, 0)),
          pl.BlockSpec((1, k, HALF), lambda i, j: (j, 0, 0)),
      ],
      out_specs=pl.BlockSpec((1, rows_blk, HALF), lambda i, j: (j, i, 0)),
      out_shape=jax.ShapeDtypeStruct((2, n, HALF), jnp.float32),
  )(h0, h1, b.reshape(1, k), W)


def _combine_mm_flat(h0, h1, b, W, rows_blk=1000):
  n = h0.shape[0]
  k, m = W.shape

  def kern(h0_ref, h1_ref, b_ref, w_ref, o_ref):
    h = jnp.concatenate([h0_ref[...], h1_ref[...]], axis=1)
    h = jnp.maximum(h + b_ref[...], 0.0)
    o_ref[...] = jnp.dot(h, w_ref[...], preferred_element_type=jnp.float32)

  return pl.pallas_call(
      kern,
      grid=(n // rows_blk,),
      in_specs=[
          pl.BlockSpec((rows_blk, HALF), lambda i: (i, 0)),
          pl.BlockSpec((rows_blk, HALF), lambda i: (i, 0)),
          pl.BlockSpec((1, k), lambda i: (0, 0)),
          pl.BlockSpec((k, m), lambda i: (0, 0)),
      ],
      out_specs=pl.BlockSpec((rows_blk, m), lambda i: (i, 0)),
      out_shape=jax.ShapeDtypeStruct((n, m), jnp.float32),
  )(h0, h1, b.reshape(1, k), W)


def _final_logsoftmax(p0, p1, b, rows_blk=1000):
  n, k = p0.shape

  def kern(p0_ref, p1_ref, b_ref, o_ref):
    v = p0_ref[...] + p1_ref[...] + b_ref[...]
    col = lax.broadcasted_iota(jnp.int32, v.shape, 1)
    valid = col < NCLASS
    vm = jnp.where(valid, v, jnp.float32(-1e30))
    mx = jnp.max(vm, axis=1, keepdims=True)
    ex = jnp.where(valid, jnp.exp(v - mx), 0.0)
    ssum = jnp.sum(ex, axis=1, keepdims=True)
    o_ref[...] = v - mx - jnp.log(ssum)

  out = pl.pallas_call(
      kern,
      grid=(n // rows_blk,),
      in_specs=[
          pl.BlockSpec((rows_blk, k), lambda i: (i, 0)),
          pl.BlockSpec((rows_blk, k), lambda i: (i, 0)),
          pl.BlockSpec((1, k), lambda i: (0, 0)),
      ],
      out_specs=pl.BlockSpec((rows_blk, k), lambda i: (i, 0)),
      out_shape=jax.ShapeDtypeStruct((n, k), jnp.float32),
  )(p0, p1, b.reshape(1, k))
  return out[:, :NCLASS]


def kernel(x, adj, W1, b1, W2, b2, W3, b3):
  padA = jnp.arange(E_PAD_A - E, dtype=jnp.int32)
  srcA = jnp.concatenate([adj[0], padA % N])
  srcAx = jnp.stack([srcA, srcA + N]).reshape(2, 16, CPT_A, CHUNK_A)
  dstA = jnp.concatenate([adj[1], N + padA % (NPAD_A - N)]).reshape(
      16, CPT_A, CHUNK_A)

  padB = jnp.arange(E_PAD_B - E, dtype=jnp.int32)
  srcB = jnp.concatenate([adj[0], padB % N]).reshape(32, CPT_B, CHUNK_B)
  dstB = jnp.concatenate([adj[1], N + padB % (NPAD_B - N)]).reshape(
      32, CPT_B, CHUNK_B)

  W1s = W1.reshape(NFEAT, 2, HALF).transpose(1, 0, 2)
  W2s = W2.reshape(NHID, 2, HALF).transpose(1, 0, 2)
  table = _mm_split(x, W1s).reshape(2 * N, HALF)
  parts = _spmm_col(table, srcAx, dstA)
  h0, h1 = parts[:N], parts[NPAD_A:NPAD_A + N]

  table = _combine_mm_split(h0, h1, b1, W2s).reshape(2 * N, HALF)
  parts = _spmm_col(table, srcAx, dstA)
  h0, h1 = parts[:N], parts[NPAD_A:NPAD_A + N]

  W3p = jnp.concatenate(
      [W3, jnp.zeros((NHID, NCLS_PAD - NCLASS), jnp.float32)], axis=1)
  b3p = jnp.concatenate([b3, jnp.zeros((NCLS_PAD - NCLASS,), jnp.float32)])
  support = _combine_mm_flat(h0, h1, b2, W3p)
  parts = _spmm_edge(support, srcB, dstB)
  q0, q1 = parts[:N], parts[NPAD_B:NPAD_B + N]

  return _final_logsoftmax(q0, q1, b3p)
